# Initial kernel scaffold; baseline (speedup 1.0000x reference)
import jax
import jax.numpy as jnp
from jax.experimental import pallas as pl


def kernel(x, edge_index, W0, b0, Wres0, bres0, W1, b1, Wres1, bres1, w_atom, b_atom):
    raise NotImplementedError("write your pallas kernel here")


# speedup vs baseline: 4.7601x; 4.7601x over previous
import functools

import jax
import jax.numpy as jnp
from jax import lax
from jax.experimental import pallas as pl
from jax.experimental.pallas import tpu as pltpu
from jax.experimental.pallas import tpu_sc as plsc

_N = 10000
_E = 320000
_D = 128
_NC = 2
_NS = 16
_NW = _NC * _NS
_EPW = _E // _NW
_K = 80
_ITERS = _EPW // _K
_RPW = _N // _NS

_mesh = plsc.VectorSubcoreMesh(core_axis_name="c", subcore_axis_name="s")


@functools.partial(
    pl.kernel,
    mesh=_mesh,
    out_type=[
        jax.ShapeDtypeStruct((_NC * _N,), jnp.float32),
        jax.ShapeDtypeStruct((_NC * _N,), jnp.float32),
    ],
    scratch_types=[
        pltpu.VMEM((_K,), jnp.int32),
        pltpu.VMEM((_K,), jnp.int32),
        pltpu.VMEM((_K,), jnp.float32),
        pltpu.VMEM((1000,), jnp.float32),
        pltpu.VMEM_SHARED((_N,), jnp.float32),
        pltpu.VMEM_SHARED((_N,), jnp.float32),
    ],
)
def _deg_kernel(src_hbm, dst_hbm, out_s, out_d,
                src_v, dst_v, ones_v, zeros_v, acc_s, acc_d):
    c = lax.axis_index("c")
    s = lax.axis_index("s")
    w = c * _NS + s

    def _fill_ones(i, carry):
        ones_v[pl.ds(i * 16, 16)] = jnp.ones((16,), jnp.float32)
        return carry

    lax.fori_loop(0, _K // 16, _fill_ones, 0)

    def _fill_zeros(i, carry):
        zeros_v[pl.ds(i * 16, 16)] = jnp.zeros((16,), jnp.float32)
        return carry

    lax.fori_loop(0, 1000 // 16, _fill_zeros, 0)

    @pl.when(s < 10)
    def _zero():
        pltpu.sync_copy(zeros_v, acc_s.at[pl.ds(s * 1000, 1000)])
        pltpu.sync_copy(zeros_v, acc_d.at[pl.ds(s * 1000, 1000)])

    plsc.subcore_barrier()

    base = w * _EPW

    def _body(j, carry):
        b = base + j * _K
        pltpu.sync_copy(src_hbm.at[pl.ds(b, _K)], src_v)
        pltpu.sync_copy(dst_hbm.at[pl.ds(b, _K)], dst_v)
        pltpu.sync_copy(ones_v, acc_s.at[src_v], add=True)
        pltpu.sync_copy(ones_v, acc_d.at[dst_v], add=True)
        return carry

    lax.fori_loop(0, _ITERS, _body, 0)
    plsc.subcore_barrier()

    @pl.when(s < 10)
    def _writeback():
        pltpu.sync_copy(acc_s.at[pl.ds(s * 1000, 1000)], zeros_v)
        pltpu.sync_copy(zeros_v, out_s.at[pl.ds(c * _N + s * 1000, 1000)])
        pltpu.sync_copy(acc_d.at[pl.ds(s * 1000, 1000)], zeros_v)
        pltpu.sync_copy(zeros_v, out_d.at[pl.ds(c * _N + s * 1000, 1000)])


@functools.partial(
    pl.kernel,
    mesh=_mesh,
    out_type=jax.ShapeDtypeStruct((_NC, _N, _D), jnp.float32),
    scratch_types=[
        pltpu.VMEM((_K,), jnp.int32),
        pltpu.VMEM((_K,), jnp.int32),
        pltpu.VMEM((_K, _D), jnp.float32),
        pltpu.VMEM((200, _D), jnp.float32),
        pltpu.VMEM_SHARED((_N, _D), jnp.float32),
        pltpu.SemaphoreType.DMA,
    ],
)
def _agg_kernel(hs_hbm, src_hbm, dst_hbm, out_hbm,
                src_v, dst_v, rows_v, zrows_v, acc, sem):
    c = lax.axis_index("c")
    s = lax.axis_index("s")
    w = c * _NS + s

    def _fill_zeros(t, carry):
        zrows_v[t // 8, pl.ds((t % 8) * 16, 16)] = jnp.zeros((16,), jnp.float32)
        return carry

    lax.fori_loop(0, 200 * 8, _fill_zeros, 0)

    @pl.when(s < 10)
    def _zero():
        def _zq(q, carry):
            pltpu.sync_copy(zrows_v, acc.at[pl.ds(s * 1000 + q * 200, 200)])
            return carry
        lax.fori_loop(0, 5, _zq, 0)

    plsc.subcore_barrier()

    base = w * _EPW

    def _body(j, carry):
        b = base + j * _K
        pltpu.sync_copy(src_hbm.at[pl.ds(b, _K)], src_v)
        pltpu.sync_copy(dst_hbm.at[pl.ds(b, _K)], dst_v)
        pltpu.async_copy(hs_hbm.at[src_v], rows_v, sem).wait()
        pltpu.sync_copy(rows_v, acc.at[dst_v], add=True)
        return carry

    lax.fori_loop(0, _ITERS, _body, 0)
    plsc.subcore_barrier()

    @pl.when(s < 10)
    def _writeback():
        def _wq(q, carry):
            r = s * 1000 + q * 200
            pltpu.sync_copy(acc.at[pl.ds(r, 200)], zrows_v)
            pltpu.sync_copy(zrows_v, out_hbm.at[c, pl.ds(r, 200)])
            return carry
        lax.fori_loop(0, 5, _wq, 0)


_RB = 1000


def _scale_body(x_ref, d_ref, o_ref):
    o_ref[...] = x_ref[...] * d_ref[...]


def _scale(x, d_col):
    return pl.pallas_call(
        _scale_body,
        grid=(_N // _RB,),
        in_specs=[
            pl.BlockSpec((_RB, _D), lambda i: (i, 0)),
            pl.BlockSpec((_RB, 1), lambda i: (i, 0)),
        ],
        out_specs=pl.BlockSpec((_RB, _D), lambda i: (i, 0)),
        out_shape=jax.ShapeDtypeStruct((_N, _D), jnp.float32),
    )(x, d_col)


def _layer_body(agg_ref, din_ref, h_ref, w_ref, b_ref, wr_ref, br_ref,
                dout_ref, o_ref, os_ref):
    agg = (agg_ref[0] + agg_ref[1]) * din_ref[...]
    new = jnp.maximum(
        jnp.dot(agg, w_ref[...], preferred_element_type=jnp.float32)
        + b_ref[...], 0.0)
    res = jnp.maximum(
        jnp.dot(h_ref[...], wr_ref[...], preferred_element_type=jnp.float32)
        + br_ref[...], 0.0)
    hn = new + res
    o_ref[...] = hn
    os_ref[...] = hn * dout_ref[...]


def _layer(agg, din, h, w, b, wr, br, dout):
    return pl.pallas_call(
        _layer_body,
        grid=(_N // _RB,),
        in_specs=[
            pl.BlockSpec((_NC, _RB, _D), lambda i: (0, i, 0)),
            pl.BlockSpec((_RB, 1), lambda i: (i, 0)),
            pl.BlockSpec((_RB, _D), lambda i: (i, 0)),
            pl.BlockSpec((_D, _D), lambda i: (0, 0)),
            pl.BlockSpec((1, _D), lambda i: (0, 0)),
            pl.BlockSpec((_D, _D), lambda i: (0, 0)),
            pl.BlockSpec((1, _D), lambda i: (0, 0)),
            pl.BlockSpec((_RB, 1), lambda i: (i, 0)),
        ],
        out_specs=[
            pl.BlockSpec((_RB, _D), lambda i: (i, 0)),
            pl.BlockSpec((_RB, _D), lambda i: (i, 0)),
        ],
        out_shape=[
            jax.ShapeDtypeStruct((_N, _D), jnp.float32),
            jax.ShapeDtypeStruct((_N, _D), jnp.float32),
        ],
    )(agg, din, h, w, b, wr, br, dout)


def _layer2_body(agg_ref, din_ref, h_ref, w_ref, b_ref, wr_ref, br_ref,
                 wa_ref, ba_ref, o_ref):
    i = pl.program_id(0)
    agg = (agg_ref[0] + agg_ref[1]) * din_ref[...]
    new = jnp.maximum(
        jnp.dot(agg, w_ref[...], preferred_element_type=jnp.float32)
        + b_ref[...], 0.0)
    res = jnp.maximum(
        jnp.dot(h_ref[...], wr_ref[...], preferred_element_type=jnp.float32)
        + br_ref[...], 0.0)
    hn = new + res
    logit = jnp.dot(hn, wa_ref[...], preferred_element_type=jnp.float32) \
        + ba_ref[...]
    wgt = jax.nn.sigmoid(logit)
    psum = jnp.sum(wgt * hn, axis=0, keepdims=True)
    pmax = jnp.max(hn, axis=0, keepdims=True)

    @pl.when(i == 0)
    def _init():
        o_ref[...] = jnp.concatenate([psum, pmax], axis=1)

    @pl.when(i > 0)
    def _acc():
        prev = o_ref[...]
        o_ref[...] = jnp.concatenate(
            [prev[:, :_D] + psum, jnp.maximum(prev[:, _D:], pmax)], axis=1)


def _layer2(agg, din, h, w, b, wr, br, wa, ba):
    return pl.pallas_call(
        _layer2_body,
        grid=(_N // _RB,),
        in_specs=[
            pl.BlockSpec((_NC, _RB, _D), lambda i: (0, i, 0)),
            pl.BlockSpec((_RB, 1), lambda i: (i, 0)),
            pl.BlockSpec((_RB, _D), lambda i: (i, 0)),
            pl.BlockSpec((_D, _D), lambda i: (0, 0)),
            pl.BlockSpec((1, _D), lambda i: (0, 0)),
            pl.BlockSpec((_D, _D), lambda i: (0, 0)),
            pl.BlockSpec((1, _D), lambda i: (0, 0)),
            pl.BlockSpec((_D, 1), lambda i: (0, 0)),
            pl.BlockSpec((1, 1), lambda i: (0, 0)),
        ],
        out_specs=pl.BlockSpec((1, 2 * _D), lambda i: (0, 0)),
        out_shape=jax.ShapeDtypeStruct((1, 2 * _D), jnp.float32),
    )(agg, din, h, w, b, wr, br, wa, ba)


def kernel(x, edge_index, W0, b0, Wres0, bres0, W1, b1, Wres1, bres1,
           w_atom, b_atom):
    src = edge_index[0]
    dst = edge_index[1]

    deg_s_part, deg_d_part = _deg_kernel(src, dst)
    di_out = lax.rsqrt(jnp.clip(deg_s_part[:_N] + deg_s_part[_N:], 1.0, None))
    di_in = lax.rsqrt(jnp.clip(deg_d_part[:_N] + deg_d_part[_N:], 1.0, None))
    di_out = di_out[:, None]
    di_in = di_in[:, None]

    hs0 = _scale(x, di_out)
    agg0 = _agg_kernel(hs0, src, dst)
    h1, hs1 = _layer(agg0, di_in, x, W0, b0[None, :], Wres0, bres0[None, :],
                     di_out)
    agg1 = _agg_kernel(hs1, src, dst)
    return _layer2(agg1, di_in, h1, W1, b1[None, :], Wres1, bres1[None, :],
                   w_atom, b_atom[None, :])


# speedup vs baseline: 9.5059x; 1.9970x over previous
import functools

import jax
import jax.numpy as jnp
from jax import lax
from jax.experimental import pallas as pl
from jax.experimental.pallas import tpu as pltpu
from jax.experimental.pallas import tpu_sc as plsc

_N = 10000
_E = 320000
_D = 128
_NC = 2
_NS = 16
_NW = _NC * _NS
_EPW = _E // _NW
_K = 80
_ITERS = _EPW // _K
_RPW = _N // _NS

_mesh = plsc.VectorSubcoreMesh(core_axis_name="c", subcore_axis_name="s")


@functools.partial(
    pl.kernel,
    mesh=_mesh,
    out_type=[
        jax.ShapeDtypeStruct((_NC * _N,), jnp.float32),
        jax.ShapeDtypeStruct((_NC * _N,), jnp.float32),
    ],
    scratch_types=[
        pltpu.VMEM((_ITERS, _K), jnp.int32),
        pltpu.VMEM((_ITERS, _K), jnp.int32),
        pltpu.VMEM((_K,), jnp.float32),
        pltpu.VMEM((1000,), jnp.float32),
        pltpu.VMEM_SHARED((_N,), jnp.float32),
        pltpu.VMEM_SHARED((_N,), jnp.float32),
    ],
)
def _deg_kernel(src_hbm, dst_hbm, out_s, out_d,
                src_v, dst_v, ones_v, zeros_v, acc_s, acc_d):
    c = lax.axis_index("c")
    s = lax.axis_index("s")
    w = c * _NS + s

    pltpu.sync_copy(src_hbm.at[w], src_v)
    pltpu.sync_copy(dst_hbm.at[w], dst_v)

    def _fill_ones(i, carry):
        ones_v[pl.ds(i * 16, 16)] = jnp.ones((16,), jnp.float32)
        return carry

    lax.fori_loop(0, _K // 16, _fill_ones, 0)

    def _fill_zeros(i, carry):
        zeros_v[pl.ds(i * 16, 16)] = jnp.zeros((16,), jnp.float32)
        return carry

    lax.fori_loop(0, 1000 // 16, _fill_zeros, 0)

    @pl.when(s < 10)
    def _zero():
        pltpu.sync_copy(zeros_v, acc_s.at[pl.ds(s * 1000, 1000)])
        pltpu.sync_copy(zeros_v, acc_d.at[pl.ds(s * 1000, 1000)])

    plsc.subcore_barrier()

    def _body(j, carry):
        pltpu.sync_copy(ones_v, acc_s.at[src_v.at[j]], add=True)
        pltpu.sync_copy(ones_v, acc_d.at[dst_v.at[j]], add=True)
        return carry

    lax.fori_loop(0, _ITERS, _body, 0)
    plsc.subcore_barrier()

    @pl.when(s < 10)
    def _writeback():
        pltpu.sync_copy(acc_s.at[pl.ds(s * 1000, 1000)], zeros_v)
        pltpu.sync_copy(zeros_v, out_s.at[pl.ds(c * _N + s * 1000, 1000)])
        pltpu.sync_copy(acc_d.at[pl.ds(s * 1000, 1000)], zeros_v)
        pltpu.sync_copy(zeros_v, out_d.at[pl.ds(c * _N + s * 1000, 1000)])


@functools.partial(
    pl.kernel,
    mesh=_mesh,
    out_type=jax.ShapeDtypeStruct((_NC, _N, _D), jnp.float32),
    scratch_types=[
        pltpu.VMEM((_EPW,), jnp.int32),
        pltpu.VMEM((_ITERS, _K), jnp.int32),
        pltpu.VMEM((_K, _D), jnp.float32),
        pltpu.VMEM((_K, _D), jnp.float32),
        pltpu.VMEM_SHARED((_N, _D), jnp.float32),
        pltpu.SemaphoreType.DMA,
        pltpu.SemaphoreType.DMA,
    ],
)
def _agg_kernel(hs_hbm, srcf_hbm, dst_hbm, zeros_hbm, out_hbm,
                src_v, dst_v, rows0, rows1, acc, sem0, sem1):
    c = lax.axis_index("c")
    s = lax.axis_index("s")
    w = c * _NS + s

    pltpu.sync_copy(srcf_hbm.at[pl.ds(w * _EPW, _EPW)], src_v)
    pltpu.sync_copy(dst_hbm.at[w], dst_v)
    pltpu.sync_copy(zeros_hbm, rows0)

    def _zq(q, carry):
        m = s + q * _NS

        @pl.when(m < _N // _K)
        def _():
            pltpu.sync_copy(rows0, acc.at[pl.ds(m * _K, _K)])
        return carry

    lax.fori_loop(0, (_N // _K + _NS - 1) // _NS, _zq, 0)
    plsc.subcore_barrier()

    def _gidx(j):
        return src_v.at[pl.ds(j * _K, _K)]

    pltpu.async_copy(hs_hbm.at[_gidx(0)], rows0, sem0)

    def _body(t, carry):
        j0 = 2 * t
        pltpu.make_async_copy(hs_hbm.at[_gidx(j0)], rows0, sem0).wait()
        pltpu.async_copy(hs_hbm.at[_gidx(j0 + 1)], rows1, sem1)
        pltpu.sync_copy(rows0, acc.at[dst_v.at[j0]], add=True)
        pltpu.make_async_copy(hs_hbm.at[_gidx(j0 + 1)], rows1, sem1).wait()
        pltpu.async_copy(hs_hbm.at[_gidx(j0 + 2)], rows0, sem0)
        pltpu.sync_copy(rows1, acc.at[dst_v.at[j0 + 1]], add=True)
        return carry

    lax.fori_loop(0, (_ITERS - 1) // 2, _body, 0)
    pltpu.make_async_copy(hs_hbm.at[_gidx(_ITERS - 1)], rows0, sem0).wait()
    pltpu.sync_copy(rows0, acc.at[dst_v.at[_ITERS - 1]], add=True)
    plsc.subcore_barrier()

    def _wq(q, carry):
        m = s + q * _NS

        @pl.when(m < _N // _K)
        def _():
            pltpu.sync_copy(acc.at[pl.ds(m * _K, _K)], rows0)
            pltpu.sync_copy(rows0, out_hbm.at[c, pl.ds(m * _K, _K)])
        return carry

    lax.fori_loop(0, (_N // _K + _NS - 1) // _NS, _wq, 0)


_RB = 1000


def _scale_body(x_ref, d_ref, o_ref):
    o_ref[...] = x_ref[...] * d_ref[...]


def _scale(x, d_col):
    return pl.pallas_call(
        _scale_body,
        grid=(_N // _RB,),
        in_specs=[
            pl.BlockSpec((_RB, _D), lambda i: (i, 0)),
            pl.BlockSpec((_RB, 1), lambda i: (i, 0)),
        ],
        out_specs=pl.BlockSpec((_RB, _D), lambda i: (i, 0)),
        out_shape=jax.ShapeDtypeStruct((_N, _D), jnp.float32),
    )(x, d_col)


def _layer_body(agg_ref, din_ref, h_ref, w_ref, b_ref, wr_ref, br_ref,
                dout_ref, o_ref, os_ref):
    agg = (agg_ref[0] + agg_ref[1]) * din_ref[...]
    new = jnp.maximum(
        jnp.dot(agg, w_ref[...], preferred_element_type=jnp.float32)
        + b_ref[...], 0.0)
    res = jnp.maximum(
        jnp.dot(h_ref[...], wr_ref[...], preferred_element_type=jnp.float32)
        + br_ref[...], 0.0)
    hn = new + res
    o_ref[...] = hn
    os_ref[...] = hn * dout_ref[...]


def _layer(agg, din, h, w, b, wr, br, dout):
    return pl.pallas_call(
        _layer_body,
        grid=(_N // _RB,),
        in_specs=[
            pl.BlockSpec((_NC, _RB, _D), lambda i: (0, i, 0)),
            pl.BlockSpec((_RB, 1), lambda i: (i, 0)),
            pl.BlockSpec((_RB, _D), lambda i: (i, 0)),
            pl.BlockSpec((_D, _D), lambda i: (0, 0)),
            pl.BlockSpec((1, _D), lambda i: (0, 0)),
            pl.BlockSpec((_D, _D), lambda i: (0, 0)),
            pl.BlockSpec((1, _D), lambda i: (0, 0)),
            pl.BlockSpec((_RB, 1), lambda i: (i, 0)),
        ],
        out_specs=[
            pl.BlockSpec((_RB, _D), lambda i: (i, 0)),
            pl.BlockSpec((_RB, _D), lambda i: (i, 0)),
        ],
        out_shape=[
            jax.ShapeDtypeStruct((_N, _D), jnp.float32),
            jax.ShapeDtypeStruct((_N, _D), jnp.float32),
        ],
    )(agg, din, h, w, b, wr, br, dout)


def _layer2_body(agg_ref, din_ref, h_ref, w_ref, b_ref, wr_ref, br_ref,
                 wa_ref, ba_ref, o_ref):
    i = pl.program_id(0)
    agg = (agg_ref[0] + agg_ref[1]) * din_ref[...]
    new = jnp.maximum(
        jnp.dot(agg, w_ref[...], preferred_element_type=jnp.float32)
        + b_ref[...], 0.0)
    res = jnp.maximum(
        jnp.dot(h_ref[...], wr_ref[...], preferred_element_type=jnp.float32)
        + br_ref[...], 0.0)
    hn = new + res
    logit = jnp.dot(hn, wa_ref[...], preferred_element_type=jnp.float32) \
        + ba_ref[...]
    wgt = jax.nn.sigmoid(logit)
    psum = jnp.sum(wgt * hn, axis=0, keepdims=True)
    pmax = jnp.max(hn, axis=0, keepdims=True)

    @pl.when(i == 0)
    def _init():
        o_ref[...] = jnp.concatenate([psum, pmax], axis=1)

    @pl.when(i > 0)
    def _acc():
        prev = o_ref[...]
        o_ref[...] = jnp.concatenate(
            [prev[:, :_D] + psum, jnp.maximum(prev[:, _D:], pmax)], axis=1)


def _layer2(agg, din, h, w, b, wr, br, wa, ba):
    return pl.pallas_call(
        _layer2_body,
        grid=(_N // _RB,),
        in_specs=[
            pl.BlockSpec((_NC, _RB, _D), lambda i: (0, i, 0)),
            pl.BlockSpec((_RB, 1), lambda i: (i, 0)),
            pl.BlockSpec((_RB, _D), lambda i: (i, 0)),
            pl.BlockSpec((_D, _D), lambda i: (0, 0)),
            pl.BlockSpec((1, _D), lambda i: (0, 0)),
            pl.BlockSpec((_D, _D), lambda i: (0, 0)),
            pl.BlockSpec((1, _D), lambda i: (0, 0)),
            pl.BlockSpec((_D, 1), lambda i: (0, 0)),
            pl.BlockSpec((1, 1), lambda i: (0, 0)),
        ],
        out_specs=pl.BlockSpec((1, 2 * _D), lambda i: (0, 0)),
        out_shape=jax.ShapeDtypeStruct((1, 2 * _D), jnp.float32),
    )(agg, din, h, w, b, wr, br, wa, ba)


def kernel(x, edge_index, W0, b0, Wres0, bres0, W1, b1, Wres1, bres1,
           w_atom, b_atom):
    srcf = edge_index[0]
    src = edge_index[0].reshape(_NW, _ITERS, _K)
    dst = edge_index[1].reshape(_NW, _ITERS, _K)
    zrows = jnp.zeros((_K, _D), jnp.float32)

    deg_s_part, deg_d_part = _deg_kernel(src, dst)
    di_out = lax.rsqrt(jnp.clip(deg_s_part[:_N] + deg_s_part[_N:], 1.0, None))
    di_in = lax.rsqrt(jnp.clip(deg_d_part[:_N] + deg_d_part[_N:], 1.0, None))
    di_out = di_out[:, None]
    di_in = di_in[:, None]

    hs0 = _scale(x, di_out)
    agg0 = _agg_kernel(hs0, srcf, dst, zrows)
    h1, hs1 = _layer(agg0, di_in, x, W0, b0[None, :], Wres0, bres0[None, :],
                     di_out)
    agg1 = _agg_kernel(hs1, srcf, dst, zrows)
    return _layer2(agg1, di_in, h1, W1, b1[None, :], Wres1, bres1[None, :],
                   w_atom, b_atom[None, :])


# speedup vs baseline: 13.2721x; 1.3962x over previous
import functools

import jax
import jax.numpy as jnp
from jax import lax
from jax.experimental import pallas as pl
from jax.experimental.pallas import tpu as pltpu
from jax.experimental.pallas import tpu_sc as plsc

_N = 10000
_E = 320000
_D = 128
_NC = 2
_NS = 16
_NW = _NC * _NS
_EPW = _E // _NW
_K = 80
_ITERS = _EPW // _K
_RPW = _N // _NS

_mesh = plsc.VectorSubcoreMesh(core_axis_name="c", subcore_axis_name="s")


@functools.partial(
    pl.kernel,
    mesh=_mesh,
    out_type=[
        jax.ShapeDtypeStruct((_NC * _N,), jnp.float32),
        jax.ShapeDtypeStruct((_NC * _N,), jnp.float32),
    ],
    scratch_types=[
        pltpu.VMEM((_ITERS, _K), jnp.int32),
        pltpu.VMEM((_ITERS, _K), jnp.int32),
        pltpu.VMEM((_K,), jnp.float32),
        pltpu.VMEM((1000,), jnp.float32),
        pltpu.VMEM_SHARED((_N,), jnp.float32),
        pltpu.VMEM_SHARED((_N,), jnp.float32),
    ],
)
def _deg_kernel(src_hbm, dst_hbm, out_s, out_d,
                src_v, dst_v, ones_v, zeros_v, acc_s, acc_d):
    c = lax.axis_index("c")
    s = lax.axis_index("s")
    w = c * _NS + s

    pltpu.sync_copy(src_hbm.at[w], src_v)
    pltpu.sync_copy(dst_hbm.at[w], dst_v)

    def _fill_ones(i, carry):
        ones_v[pl.ds(i * 16, 16)] = jnp.ones((16,), jnp.float32)
        return carry

    lax.fori_loop(0, _K // 16, _fill_ones, 0)

    def _fill_zeros(i, carry):
        zeros_v[pl.ds(i * 16, 16)] = jnp.zeros((16,), jnp.float32)
        return carry

    lax.fori_loop(0, 1000 // 16, _fill_zeros, 0)

    @pl.when(s < 10)
    def _zero():
        pltpu.sync_copy(zeros_v, acc_s.at[pl.ds(s * 1000, 1000)])
        pltpu.sync_copy(zeros_v, acc_d.at[pl.ds(s * 1000, 1000)])

    plsc.subcore_barrier()

    def _body(j, carry):
        pltpu.sync_copy(ones_v, acc_s.at[src_v.at[j]], add=True)
        pltpu.sync_copy(ones_v, acc_d.at[dst_v.at[j]], add=True)
        return carry

    lax.fori_loop(0, _ITERS, _body, 0)
    plsc.subcore_barrier()

    @pl.when(s < 10)
    def _writeback():
        pltpu.sync_copy(acc_s.at[pl.ds(s * 1000, 1000)], zeros_v)
        pltpu.sync_copy(zeros_v, out_s.at[pl.ds(c * _N + s * 1000, 1000)])
        pltpu.sync_copy(acc_d.at[pl.ds(s * 1000, 1000)], zeros_v)
        pltpu.sync_copy(zeros_v, out_d.at[pl.ds(c * _N + s * 1000, 1000)])


@functools.partial(
    pl.kernel,
    mesh=_mesh,
    out_type=jax.ShapeDtypeStruct((_NC, _N, _D), jnp.float32),
    scratch_types=[
        pltpu.VMEM((_EPW,), jnp.int32),
        pltpu.VMEM((_K,), jnp.int32),
        pltpu.VMEM((_K,), jnp.int32),
        pltpu.VMEM((_K,), jnp.int32),
        pltpu.VMEM((_K, _D), jnp.float32),
        pltpu.VMEM((_K, _D), jnp.float32),
        pltpu.VMEM((_K, _D), jnp.float32),
        pltpu.VMEM_SHARED((_N, _D), jnp.float32),
        pltpu.SemaphoreType.DMA,
        pltpu.SemaphoreType.DMA,
        pltpu.SemaphoreType.DMA,
        pltpu.SemaphoreType.DMA,
        pltpu.SemaphoreType.DMA,
        pltpu.SemaphoreType.DMA,
        pltpu.SemaphoreType.DMA,
        pltpu.SemaphoreType.DMA,
        pltpu.SemaphoreType.DMA,
    ],
)
def _agg_kernel(hs_hbm, srcf_hbm, dstf_hbm, zeros_hbm, out_hbm,
                src_v, st0, st1, st2, rw0, rw1, rw2, acc,
                g0, g1, g2, d0, d1, d2, t0, t1, t2):
    c = lax.axis_index("c")
    s = lax.axis_index("s")
    w = c * _NS + s
    stage = (st0, st1, st2)
    rows = (rw0, rw1, rw2)
    g = (g0, g1, g2)
    d = (d0, d1, d2)
    t = (t0, t1, t2)

    pltpu.sync_copy(srcf_hbm.at[pl.ds(w * _EPW, _EPW)], src_v)
    pltpu.sync_copy(zeros_hbm, rw0)

    def _zq(q, carry):
        m = s + q * _NS

        @pl.when(m < _N // _K)
        def _():
            pltpu.sync_copy(rw0, acc.at[pl.ds(m * _K, _K)])
        return carry

    lax.fori_loop(0, (_N // _K + _NS - 1) // _NS, _zq, 0)
    plsc.subcore_barrier()

    def _gather_args(j, r):
        return hs_hbm.at[src_v.at[pl.ds(j * _K, _K)]], rows[r], g[r]

    def _didx_args(j, r):
        return dstf_hbm.at[pl.ds(w * _EPW + j * _K, _K)], stage[r], d[r]

    def _scat_args(r):
        return rows[r], acc.at[stage[r]], t[r]

    for jj in (0, 1):
        pltpu.async_copy(*_gather_args(jj, jj))
        pltpu.async_copy(*_didx_args(jj, jj))

    def _substep(j, r):
        r2 = (r + 2) % 3

        @pl.when(j >= 1)
        def _():
            pltpu.make_async_copy(*_scat_args(r2)).wait()
        pltpu.async_copy(*_gather_args(j + 2, r2))
        pltpu.async_copy(*_didx_args(j + 2, r2))
        pltpu.make_async_copy(*_gather_args(j, r)).wait()
        pltpu.make_async_copy(*_didx_args(j, r)).wait()
        pltpu.async_copy(*_scat_args(r), add=True)

    def _body(tt, carry):
        j0 = 3 * tt
        _substep(j0, 0)
        _substep(j0 + 1, 1)
        _substep(j0 + 2, 2)
        return carry

    lax.fori_loop(0, (_ITERS - 2) // 3, _body, 0)
    pltpu.make_async_copy(*_scat_args(2)).wait()
    pltpu.make_async_copy(*_gather_args(_ITERS - 2, 0)).wait()
    pltpu.make_async_copy(*_didx_args(_ITERS - 2, 0)).wait()
    pltpu.async_copy(*_scat_args(0), add=True)
    pltpu.make_async_copy(*_gather_args(_ITERS - 1, 1)).wait()
    pltpu.make_async_copy(*_didx_args(_ITERS - 1, 1)).wait()
    pltpu.async_copy(*_scat_args(1), add=True)
    pltpu.make_async_copy(*_scat_args(0)).wait()
    pltpu.make_async_copy(*_scat_args(1)).wait()
    plsc.subcore_barrier()

    def _wq(q, carry):
        m = s + q * _NS

        @pl.when(m < _N // _K)
        def _():
            pltpu.sync_copy(acc.at[pl.ds(m * _K, _K)], rw0)
            pltpu.sync_copy(rw0, out_hbm.at[c, pl.ds(m * _K, _K)])
        return carry

    lax.fori_loop(0, (_N // _K + _NS - 1) // _NS, _wq, 0)


_RB = 1000


def _scale_body(x_ref, d_ref, o_ref):
    o_ref[...] = x_ref[...] * d_ref[...]


def _scale(x, d_col):
    return pl.pallas_call(
        _scale_body,
        grid=(_N // _RB,),
        in_specs=[
            pl.BlockSpec((_RB, _D), lambda i: (i, 0)),
            pl.BlockSpec((_RB, 1), lambda i: (i, 0)),
        ],
        out_specs=pl.BlockSpec((_RB, _D), lambda i: (i, 0)),
        out_shape=jax.ShapeDtypeStruct((_N, _D), jnp.float32),
    )(x, d_col)


def _layer_body(agg_ref, din_ref, h_ref, w_ref, b_ref, wr_ref, br_ref,
                dout_ref, o_ref, os_ref):
    agg = (agg_ref[0] + agg_ref[1]) * din_ref[...]
    new = jnp.maximum(
        jnp.dot(agg, w_ref[...], preferred_element_type=jnp.float32)
        + b_ref[...], 0.0)
    res = jnp.maximum(
        jnp.dot(h_ref[...], wr_ref[...], preferred_element_type=jnp.float32)
        + br_ref[...], 0.0)
    hn = new + res
    o_ref[...] = hn
    os_ref[...] = hn * dout_ref[...]


def _layer(agg, din, h, w, b, wr, br, dout):
    return pl.pallas_call(
        _layer_body,
        grid=(_N // _RB,),
        in_specs=[
            pl.BlockSpec((_NC, _RB, _D), lambda i: (0, i, 0)),
            pl.BlockSpec((_RB, 1), lambda i: (i, 0)),
            pl.BlockSpec((_RB, _D), lambda i: (i, 0)),
            pl.BlockSpec((_D, _D), lambda i: (0, 0)),
            pl.BlockSpec((1, _D), lambda i: (0, 0)),
            pl.BlockSpec((_D, _D), lambda i: (0, 0)),
            pl.BlockSpec((1, _D), lambda i: (0, 0)),
            pl.BlockSpec((_RB, 1), lambda i: (i, 0)),
        ],
        out_specs=[
            pl.BlockSpec((_RB, _D), lambda i: (i, 0)),
            pl.BlockSpec((_RB, _D), lambda i: (i, 0)),
        ],
        out_shape=[
            jax.ShapeDtypeStruct((_N, _D), jnp.float32),
            jax.ShapeDtypeStruct((_N, _D), jnp.float32),
        ],
    )(agg, din, h, w, b, wr, br, dout)


def _layer2_body(agg_ref, din_ref, h_ref, w_ref, b_ref, wr_ref, br_ref,
                 wa_ref, ba_ref, o_ref):
    i = pl.program_id(0)
    agg = (agg_ref[0] + agg_ref[1]) * din_ref[...]
    new = jnp.maximum(
        jnp.dot(agg, w_ref[...], preferred_element_type=jnp.float32)
        + b_ref[...], 0.0)
    res = jnp.maximum(
        jnp.dot(h_ref[...], wr_ref[...], preferred_element_type=jnp.float32)
        + br_ref[...], 0.0)
    hn = new + res
    logit = jnp.dot(hn, wa_ref[...], preferred_element_type=jnp.float32) \
        + ba_ref[...]
    wgt = jax.nn.sigmoid(logit)
    psum = jnp.sum(wgt * hn, axis=0, keepdims=True)
    pmax = jnp.max(hn, axis=0, keepdims=True)

    @pl.when(i == 0)
    def _init():
        o_ref[...] = jnp.concatenate([psum, pmax], axis=1)

    @pl.when(i > 0)
    def _acc():
        prev = o_ref[...]
        o_ref[...] = jnp.concatenate(
            [prev[:, :_D] + psum, jnp.maximum(prev[:, _D:], pmax)], axis=1)


def _layer2(agg, din, h, w, b, wr, br, wa, ba):
    return pl.pallas_call(
        _layer2_body,
        grid=(_N // _RB,),
        in_specs=[
            pl.BlockSpec((_NC, _RB, _D), lambda i: (0, i, 0)),
            pl.BlockSpec((_RB, 1), lambda i: (i, 0)),
            pl.BlockSpec((_RB, _D), lambda i: (i, 0)),
            pl.BlockSpec((_D, _D), lambda i: (0, 0)),
            pl.BlockSpec((1, _D), lambda i: (0, 0)),
            pl.BlockSpec((_D, _D), lambda i: (0, 0)),
            pl.BlockSpec((1, _D), lambda i: (0, 0)),
            pl.BlockSpec((_D, 1), lambda i: (0, 0)),
            pl.BlockSpec((1, 1), lambda i: (0, 0)),
        ],
        out_specs=pl.BlockSpec((1, 2 * _D), lambda i: (0, 0)),
        out_shape=jax.ShapeDtypeStruct((1, 2 * _D), jnp.float32),
    )(agg, din, h, w, b, wr, br, wa, ba)


def kernel(x, edge_index, W0, b0, Wres0, bres0, W1, b1, Wres1, bres1,
           w_atom, b_atom):
    srcf = edge_index[0]
    dstf = edge_index[1]
    src = edge_index[0].reshape(_NW, _ITERS, _K)
    dst = edge_index[1].reshape(_NW, _ITERS, _K)
    zrows = jnp.zeros((_K, _D), jnp.float32)

    deg_s_part, deg_d_part = _deg_kernel(src, dst)
    di_out = lax.rsqrt(jnp.clip(deg_s_part[:_N] + deg_s_part[_N:], 1.0, None))
    di_in = lax.rsqrt(jnp.clip(deg_d_part[:_N] + deg_d_part[_N:], 1.0, None))
    di_out = di_out[:, None]
    di_in = di_in[:, None]

    hs0 = _scale(x, di_out)
    agg0 = _agg_kernel(hs0, srcf, dstf, zrows)
    h1, hs1 = _layer(agg0, di_in, x, W0, b0[None, :], Wres0, bres0[None, :],
                     di_out)
    agg1 = _agg_kernel(hs1, srcf, dstf, zrows)
    return _layer2(agg1, di_in, h1, W1, b1[None, :], Wres1, bres1[None, :],
                   w_atom, b_atom[None, :])


# speedup vs baseline: 13.9376x; 1.0501x over previous
import functools

import jax
import jax.numpy as jnp
from jax import lax
from jax.experimental import pallas as pl
from jax.experimental.pallas import tpu as pltpu
from jax.experimental.pallas import tpu_sc as plsc

_N = 10000
_E = 320000
_D = 128
_NC = 2
_NS = 16
_NW = _NC * _NS
_EPW = _E // _NW
_K = 80
_ITERS = _EPW // _K
_RPW = _N // _NS

_mesh = plsc.VectorSubcoreMesh(core_axis_name="c", subcore_axis_name="s")


@functools.partial(
    pl.kernel,
    mesh=_mesh,
    out_type=[
        jax.ShapeDtypeStruct((_NC * _N,), jnp.float32),
        jax.ShapeDtypeStruct((_NC * _N,), jnp.float32),
    ],
    scratch_types=[
        pltpu.VMEM((_ITERS, _K), jnp.int32),
        pltpu.VMEM((_ITERS, _K), jnp.int32),
        pltpu.VMEM((_K,), jnp.float32),
        pltpu.VMEM((1000,), jnp.float32),
        pltpu.VMEM_SHARED((_N,), jnp.float32),
        pltpu.VMEM_SHARED((_N,), jnp.float32),
        pltpu.SemaphoreType.DMA,
        pltpu.SemaphoreType.DMA,
        pltpu.SemaphoreType.DMA,
        pltpu.SemaphoreType.DMA,
    ],
)
def _deg_kernel(src_hbm, dst_hbm, out_s, out_d,
                src_v, dst_v, ones_v, zeros_v, acc_s, acc_d,
                sa0, sa1, sb0, sb1):
    c = lax.axis_index("c")
    s = lax.axis_index("s")
    w = c * _NS + s

    pltpu.sync_copy(src_hbm.at[w], src_v)
    pltpu.sync_copy(dst_hbm.at[w], dst_v)

    def _fill_ones(i, carry):
        ones_v[pl.ds(i * 16, 16)] = jnp.ones((16,), jnp.float32)
        return carry

    lax.fori_loop(0, _K // 16, _fill_ones, 0)

    def _fill_zeros(i, carry):
        zeros_v[pl.ds(i * 16, 16)] = jnp.zeros((16,), jnp.float32)
        return carry

    lax.fori_loop(0, 1000 // 16, _fill_zeros, 0)

    @pl.when(s < 10)
    def _zero():
        pltpu.sync_copy(zeros_v, acc_s.at[pl.ds(s * 1000, 1000)])
        pltpu.sync_copy(zeros_v, acc_d.at[pl.ds(s * 1000, 1000)])

    plsc.subcore_barrier()

    def _sa(j, sem):
        return ones_v, acc_s.at[src_v.at[j]], sem

    def _sb(j, sem):
        return ones_v, acc_d.at[dst_v.at[j]], sem

    pltpu.async_copy(*_sa(0, sa0), add=True)
    pltpu.async_copy(*_sb(0, sb0), add=True)
    pltpu.async_copy(*_sa(1, sa1), add=True)
    pltpu.async_copy(*_sb(1, sb1), add=True)

    def _body(tt, carry):
        j0 = 2 * tt
        pltpu.make_async_copy(*_sa(j0 - 2, sa0)).wait()
        pltpu.async_copy(*_sa(j0, sa0), add=True)
        pltpu.make_async_copy(*_sb(j0 - 2, sb0)).wait()
        pltpu.async_copy(*_sb(j0, sb0), add=True)
        pltpu.make_async_copy(*_sa(j0 - 1, sa1)).wait()
        pltpu.async_copy(*_sa(j0 + 1, sa1), add=True)
        pltpu.make_async_copy(*_sb(j0 - 1, sb1)).wait()
        pltpu.async_copy(*_sb(j0 + 1, sb1), add=True)
        return carry

    lax.fori_loop(1, (_ITERS - 1) // 2, _body, 0)
    pltpu.make_async_copy(*_sa(_ITERS - 3, sa0)).wait()
    pltpu.async_copy(*_sa(_ITERS - 1, sa0), add=True)
    pltpu.make_async_copy(*_sb(_ITERS - 3, sb0)).wait()
    pltpu.async_copy(*_sb(_ITERS - 1, sb0), add=True)
    pltpu.make_async_copy(*_sa(_ITERS - 2, sa1)).wait()
    pltpu.make_async_copy(*_sb(_ITERS - 2, sb1)).wait()
    pltpu.make_async_copy(*_sa(_ITERS - 1, sa0)).wait()
    pltpu.make_async_copy(*_sb(_ITERS - 1, sb0)).wait()
    plsc.subcore_barrier()

    @pl.when(s < 10)
    def _writeback():
        pltpu.sync_copy(acc_s.at[pl.ds(s * 1000, 1000)], zeros_v)
        pltpu.sync_copy(zeros_v, out_s.at[pl.ds(c * _N + s * 1000, 1000)])
        pltpu.sync_copy(acc_d.at[pl.ds(s * 1000, 1000)], zeros_v)
        pltpu.sync_copy(zeros_v, out_d.at[pl.ds(c * _N + s * 1000, 1000)])


@functools.partial(
    pl.kernel,
    mesh=_mesh,
    out_type=jax.ShapeDtypeStruct((_NC, _N, _D), jnp.float32),
    scratch_types=[
        pltpu.VMEM((_EPW,), jnp.int32),
        pltpu.VMEM((_K,), jnp.int32),
        pltpu.VMEM((_K,), jnp.int32),
        pltpu.VMEM((_K,), jnp.int32),
        pltpu.VMEM((_K, _D), jnp.float32),
        pltpu.VMEM((_K, _D), jnp.float32),
        pltpu.VMEM((_K, _D), jnp.float32),
        pltpu.VMEM_SHARED((_N, _D), jnp.float32),
        pltpu.SemaphoreType.DMA,
        pltpu.SemaphoreType.DMA,
        pltpu.SemaphoreType.DMA,
        pltpu.SemaphoreType.DMA,
        pltpu.SemaphoreType.DMA,
        pltpu.SemaphoreType.DMA,
        pltpu.SemaphoreType.DMA,
        pltpu.SemaphoreType.DMA,
        pltpu.SemaphoreType.DMA,
    ],
)
def _agg_kernel(hs_hbm, srcf_hbm, dstf_hbm, zeros_hbm, out_hbm,
                src_v, st0, st1, st2, rw0, rw1, rw2, acc,
                g0, g1, g2, d0, d1, d2, t0, t1, t2):
    c = lax.axis_index("c")
    s = lax.axis_index("s")
    w = c * _NS + s
    stage = (st0, st1, st2)
    rows = (rw0, rw1, rw2)
    g = (g0, g1, g2)
    d = (d0, d1, d2)
    t = (t0, t1, t2)

    pltpu.sync_copy(srcf_hbm.at[pl.ds(w * _EPW, _EPW)], src_v)
    pltpu.sync_copy(zeros_hbm, rw0)

    def _zq(q, carry):
        m = s + q * _NS

        @pl.when(m < _N // _K)
        def _():
            pltpu.sync_copy(rw0, acc.at[pl.ds(m * _K, _K)])
        return carry

    lax.fori_loop(0, (_N // _K + _NS - 1) // _NS, _zq, 0)
    plsc.subcore_barrier()

    def _gather_args(j, r):
        return hs_hbm.at[src_v.at[pl.ds(j * _K, _K)]], rows[r], g[r]

    def _didx_args(j, r):
        return dstf_hbm.at[pl.ds(w * _EPW + j * _K, _K)], stage[r], d[r]

    def _scat_args(r):
        return rows[r], acc.at[stage[r]], t[r]

    for jj in (0, 1):
        pltpu.async_copy(*_gather_args(jj, jj))
        pltpu.async_copy(*_didx_args(jj, jj))

    def _substep(j, r):
        r2 = (r + 2) % 3

        @pl.when(j >= 1)
        def _():
            pltpu.make_async_copy(*_scat_args(r2)).wait()
        pltpu.async_copy(*_gather_args(j + 2, r2))
        pltpu.async_copy(*_didx_args(j + 2, r2))
        pltpu.make_async_copy(*_gather_args(j, r)).wait()
        pltpu.make_async_copy(*_didx_args(j, r)).wait()
        pltpu.async_copy(*_scat_args(r), add=True)

    def _body(tt, carry):
        j0 = 3 * tt
        _substep(j0, 0)
        _substep(j0 + 1, 1)
        _substep(j0 + 2, 2)
        return carry

    lax.fori_loop(0, (_ITERS - 2) // 3, _body, 0)
    pltpu.make_async_copy(*_scat_args(2)).wait()
    pltpu.make_async_copy(*_gather_args(_ITERS - 2, 0)).wait()
    pltpu.make_async_copy(*_didx_args(_ITERS - 2, 0)).wait()
    pltpu.async_copy(*_scat_args(0), add=True)
    pltpu.make_async_copy(*_gather_args(_ITERS - 1, 1)).wait()
    pltpu.make_async_copy(*_didx_args(_ITERS - 1, 1)).wait()
    pltpu.async_copy(*_scat_args(1), add=True)
    pltpu.make_async_copy(*_scat_args(0)).wait()
    pltpu.make_async_copy(*_scat_args(1)).wait()
    plsc.subcore_barrier()

    def _wq(q, carry):
        m = s + q * _NS

        @pl.when(m < _N // _K)
        def _():
            pltpu.sync_copy(acc.at[pl.ds(m * _K, _K)], rw0)
            pltpu.sync_copy(rw0, out_hbm.at[c, pl.ds(m * _K, _K)])
        return carry

    lax.fori_loop(0, (_N // _K + _NS - 1) // _NS, _wq, 0)


_RB = 1000


_FULL = pl.BlockSpec((_D, _D), lambda i: (0, 0))
_BIAS = pl.BlockSpec((1, _D), lambda i: (0, 0))
_ROWS = pl.BlockSpec((_RB, _D), lambda i: (i, 0))
_COL = pl.BlockSpec((_RB, 1), lambda i: (i, 0))
_AGG = pl.BlockSpec((_NC, _RB, _D), lambda i: (0, i, 0))


def _relu_mm(xb, w_ref, b_ref):
    return jnp.maximum(
        jnp.dot(xb, w_ref[...], preferred_element_type=jnp.float32)
        + b_ref[...], 0.0)


def _prep_body(x_ref, dout_ref, wr_ref, br_ref, hs_ref, res_ref):
    xb = x_ref[...]
    hs_ref[...] = xb * dout_ref[...]
    res_ref[...] = _relu_mm(xb, wr_ref, br_ref)


def _prep(x, dout, wr, br):
    return pl.pallas_call(
        _prep_body,
        grid=(_N // _RB,),
        in_specs=[_ROWS, _COL, _FULL, _BIAS],
        out_specs=[_ROWS, _ROWS],
        out_shape=[
            jax.ShapeDtypeStruct((_N, _D), jnp.float32),
            jax.ShapeDtypeStruct((_N, _D), jnp.float32),
        ],
    )(x, dout, wr, br)


def _layer1_body(agg_ref, din_ref, res1_ref, w_ref, b_ref, dout_ref,
                 wr2_ref, br2_ref, hs1_ref, res2_ref):
    agg = (agg_ref[0] + agg_ref[1]) * din_ref[...]
    h1 = _relu_mm(agg, w_ref, b_ref) + res1_ref[...]
    hs1_ref[...] = h1 * dout_ref[...]
    res2_ref[...] = _relu_mm(h1, wr2_ref, br2_ref)


def _layer1(agg, din, res1, w, b, dout, wr2, br2):
    return pl.pallas_call(
        _layer1_body,
        grid=(_N // _RB,),
        in_specs=[_AGG, _COL, _ROWS, _FULL, _BIAS, _COL, _FULL, _BIAS],
        out_specs=[_ROWS, _ROWS],
        out_shape=[
            jax.ShapeDtypeStruct((_N, _D), jnp.float32),
            jax.ShapeDtypeStruct((_N, _D), jnp.float32),
        ],
    )(agg, din, res1, w, b, dout, wr2, br2)


def _final_body(agg_ref, din_ref, res2_ref, w_ref, b_ref,
                wa_ref, ba_ref, o_ref):
    i = pl.program_id(0)
    agg = (agg_ref[0] + agg_ref[1]) * din_ref[...]
    h2 = _relu_mm(agg, w_ref, b_ref) + res2_ref[...]
    logit = jnp.dot(h2, wa_ref[...], preferred_element_type=jnp.float32) \
        + ba_ref[...]
    wgt = jax.nn.sigmoid(logit)
    psum = jnp.sum(wgt * h2, axis=0, keepdims=True)
    pmax = jnp.max(h2, axis=0, keepdims=True)

    @pl.when(i == 0)
    def _init():
        o_ref[...] = jnp.concatenate([psum, pmax], axis=1)

    @pl.when(i > 0)
    def _acc():
        prev = o_ref[...]
        o_ref[...] = jnp.concatenate(
            [prev[:, :_D] + psum, jnp.maximum(prev[:, _D:], pmax)], axis=1)


def _final(agg, din, res2, w, b, wa, ba):
    return pl.pallas_call(
        _final_body,
        grid=(_N // _RB,),
        in_specs=[_AGG, _COL, _ROWS, _FULL, _BIAS,
                  pl.BlockSpec((_D, 1), lambda i: (0, 0)),
                  pl.BlockSpec((1, 1), lambda i: (0, 0))],
        out_specs=pl.BlockSpec((1, 2 * _D), lambda i: (0, 0)),
        out_shape=jax.ShapeDtypeStruct((1, 2 * _D), jnp.float32),
    )(agg, din, res2, w, b, wa, ba)


def kernel(x, edge_index, W0, b0, Wres0, bres0, W1, b1, Wres1, bres1,
           w_atom, b_atom):
    srcf = edge_index[0]
    dstf = edge_index[1]
    src = edge_index[0].reshape(_NW, _ITERS, _K)
    dst = edge_index[1].reshape(_NW, _ITERS, _K)
    zrows = jnp.zeros((_K, _D), jnp.float32)

    deg_s_part, deg_d_part = _deg_kernel(src, dst)
    di_out = lax.rsqrt(jnp.clip(deg_s_part[:_N] + deg_s_part[_N:], 1.0, None))
    di_in = lax.rsqrt(jnp.clip(deg_d_part[:_N] + deg_d_part[_N:], 1.0, None))
    di_out = di_out[:, None]
    di_in = di_in[:, None]

    hs0, res1 = _prep(x, di_out, Wres0, bres0[None, :])
    agg0 = _agg_kernel(hs0, srcf, dstf, zrows)
    hs1, res2 = _layer1(agg0, di_in, res1, W0, b0[None, :], di_out,
                        Wres1, bres1[None, :])
    agg1 = _agg_kernel(hs1, srcf, dstf, zrows)
    return _final(agg1, di_in, res2, W1, b1[None, :],
                  w_atom, b_atom[None, :])
